# baseline (speedup 1.0000x reference)
import functools

import jax
import jax.numpy as jnp
from jax import lax
from jax.experimental import pallas as pl
from jax.experimental.pallas import tpu as pltpu
from jax.experimental.pallas import tpu_sc as plsc

NBINS = 15
NWORKERS = 32
HISTW = 256


def _rowmax_argmax(xt, s_iota):
    v3 = xt.reshape(16, 8, 128)
    v = v3
    while v.shape[0] > 1:
        h = v.shape[0] // 2
        v = jnp.maximum(v[:h], v[h:])
    v = v[0]
    for k in (4, 2, 1):
        v = jnp.maximum(v, jnp.roll(v, k, axis=0))
    m = v3 == v[None, :, :]
    ws = [jnp.where(m[j], float(j), 3.0e4) for j in range(16)]
    while len(ws) > 1:
        h = len(ws) // 2
        ws = [jnp.minimum(ws[i], ws[i + h]) for i in range(h)]
    w = ws[0] * 8.0 + s_iota
    for k in (4, 2, 1):
        w = jnp.minimum(w, jnp.roll(w, k, axis=0))
    return v[0:1, :], w[0:1, :]


def _tc_rows_block(x_ref, t_ref, d_ref, b_ref):
    x = x_ref[:]
    R, C = x.shape
    T = R // 128
    s_iota = lax.broadcasted_iota(jnp.int32, (8, 128), 0).astype(jnp.float32)
    confs = []
    preds = []
    for t in range(T):
        xt = x[t * 128:(t + 1) * 128, :].T
        c_t, p_t = _rowmax_argmax(xt, s_iota)
        confs.append(c_t)
        preds.append(p_t)
    conf = jnp.concatenate(confs, axis=0)
    pred = jnp.concatenate(preds, axis=0)
    tgt = t_ref[0, 0, :].reshape(T, 128).astype(jnp.float32)
    correct = (pred == tgt).astype(jnp.float32)
    d_ref[0, :, :] = correct - conf
    b_ref[0, :, :] = jnp.clip(
        jnp.floor(conf * NBINS).astype(jnp.int32), 0, NBINS - 1)


def _sc_hist(d_hbm, b_hbm, out_hbm, d_v, b_v, hist_v):
    chunk = d_v.shape[0]
    wid = lax.axis_index("s") * 2 + lax.axis_index("c")
    base = wid * chunk
    pltpu.sync_copy(d_hbm.at[pl.ds(base, chunk)], d_v)
    pltpu.sync_copy(b_hbm.at[pl.ds(base, chunk)], b_v)
    zeros16 = jnp.zeros((16,), jnp.float32)
    for r in range(HISTW // 16):
        hist_v[pl.ds(r * 16, 16)] = zeros16
    def body(i, carries):
        off = i * 16
        dv = d_v[pl.ds(off, 16)]
        bv = b_v[pl.ds(off, 16)]
        return tuple(c + jnp.where(bv == b, dv, 0.0)
                     for b, c in enumerate(carries))

    zeros16 = jnp.zeros((16,), jnp.float32)
    parts = lax.fori_loop(0, chunk // 16, body, (zeros16,) * NBINS)
    for b in range(NBINS):
        hist_v[pl.ds(b * 16, 16)] = parts[b]
    pltpu.sync_copy(hist_v, out_hbm.at[pl.ds(wid * HISTW, HISTW)])


def _tc_finish(h_ref, o_ref):
    h = h_ref[:]
    s = jnp.sum(h, axis=0)
    loss = 0.0
    for b in range(NBINS):
        loss = loss + jnp.abs(jnp.sum(s[b * 16:(b + 1) * 16]))
    o_ref[:, :] = jnp.full((1, 128), loss, jnp.float32)


def kernel(inputs, targets):
    N, C = inputs.shape
    R = 32768
    CH = 2
    Nc = N // CH
    NBc = Nc // R
    T = R // 128
    tgt = targets.astype(jnp.int32)
    mesh = plsc.VectorSubcoreMesh(core_axis_name="c", subcore_axis_name="s")
    chunk = Nc // NWORKERS
    sc_fn = functools.partial(
        pl.kernel,
        out_type=jax.ShapeDtypeStruct((NWORKERS * HISTW,), jnp.float32),
        mesh=mesh,
        scratch_types=[
            pltpu.VMEM((chunk,), jnp.float32),
            pltpu.VMEM((chunk,), jnp.int32),
            pltpu.VMEM((HISTW,), jnp.float32),
        ],
    )(_sc_hist)

    hists = []
    for c in range(CH):
        xc = lax.slice_in_dim(inputs, c * Nc, (c + 1) * Nc, axis=0)
        tc = lax.slice_in_dim(tgt, c * Nc, (c + 1) * Nc, axis=0).reshape(NBc, 1, R)
        d_arr, b_arr = pl.pallas_call(
            _tc_rows_block,
            grid=(NBc,),
            in_specs=[
                pl.BlockSpec((R, C), lambda i: (i, 0)),
                pl.BlockSpec((1, 1, R), lambda i: (i, 0, 0)),
            ],
            out_specs=[
                pl.BlockSpec((1, T, 128), lambda i: (i, 0, 0)),
                pl.BlockSpec((1, T, 128), lambda i: (i, 0, 0)),
            ],
            out_shape=[
                jax.ShapeDtypeStruct((NBc, T, 128), jnp.float32),
                jax.ShapeDtypeStruct((NBc, T, 128), jnp.int32),
            ],
        )(xc, tc)
        hists.append(sc_fn(d_arr.reshape(Nc), b_arr.reshape(Nc)))

    h = jnp.concatenate(hists).reshape(CH * NWORKERS, HISTW)
    out = pl.pallas_call(
        _tc_finish,
        out_specs=pl.BlockSpec((1, 128), lambda: (0, 0)),
        out_shape=jax.ShapeDtypeStruct((1, 128), jnp.float32),
    )(h)
    return (out[0, 0] / N).reshape(())


# speedup vs baseline: 2.5461x; 2.5461x over previous
import functools

import jax
import jax.numpy as jnp
from jax import lax
from jax.experimental import pallas as pl
from jax.experimental.pallas import tpu as pltpu
from jax.experimental.pallas import tpu_sc as plsc

NBINS = 15
NWORKERS = 32
HISTW = 256


def _rowmax_argmax(xt, s_iota):
    v3 = xt.reshape(16, 8, 128)
    v = v3
    while v.shape[0] > 1:
        h = v.shape[0] // 2
        v = jnp.maximum(v[:h], v[h:])
    v = v[0]
    for k in (4, 2, 1):
        v = jnp.maximum(v, jnp.roll(v, k, axis=0))
    m = v3 == v[None, :, :]
    ws = [jnp.where(m[j], float(j), 3.0e4) for j in range(16)]
    while len(ws) > 1:
        h = len(ws) // 2
        ws = [jnp.minimum(ws[i], ws[i + h]) for i in range(h)]
    w = ws[0] * 8.0 + s_iota
    for k in (4, 2, 1):
        w = jnp.minimum(w, jnp.roll(w, k, axis=0))
    return v[0:1, :], w[0:1, :]


def _tc_rows_block(x_ref, t_ref, d_ref, b_ref):
    x = x_ref[:]
    R, C = x.shape
    T = R // 128
    s_iota = lax.broadcasted_iota(jnp.int32, (8, 128), 0).astype(jnp.float32)
    confs = []
    preds = []
    for t in range(T):
        xt = x[t * 128:(t + 1) * 128, :].T
        c_t, p_t = _rowmax_argmax(xt, s_iota)
        confs.append(c_t)
        preds.append(p_t)
    conf = jnp.concatenate(confs, axis=0)
    pred = jnp.concatenate(preds, axis=0)
    tgt = t_ref[0, 0, :].reshape(T, 128).astype(jnp.float32)
    correct = (pred == tgt).astype(jnp.float32)
    d_ref[0, :, :] = correct - conf
    b_ref[0, :, :] = jnp.clip(
        jnp.floor(conf * NBINS).astype(jnp.int32), 0, NBINS - 1)


def _sc_hist(d_hbm, b_hbm, out_hbm, d_v, b_v, hist_v):
    chunk = d_v.shape[0]
    wid = lax.axis_index("s") * 2 + lax.axis_index("c")
    base = wid * chunk
    pltpu.sync_copy(d_hbm.at[pl.ds(base, chunk)], d_v)
    pltpu.sync_copy(b_hbm.at[pl.ds(base, chunk)], b_v)
    zeros16 = jnp.zeros((16,), jnp.float32)
    for r in range(HISTW // 16):
        hist_v[pl.ds(r * 16, 16)] = zeros16
    def body(i, carries):
        off = i * 16
        dv = d_v[pl.ds(off, 16)]
        bv = b_v[pl.ds(off, 16)]
        return tuple(c + jnp.where(bv == b, dv, 0.0)
                     for b, c in enumerate(carries))

    zeros16 = jnp.zeros((16,), jnp.float32)
    parts = lax.fori_loop(0, chunk // 16, body, (zeros16,) * NBINS)
    for b in range(NBINS):
        hist_v[pl.ds(b * 16, 16)] = parts[b]
    pltpu.sync_copy(hist_v, out_hbm.at[pl.ds(wid * HISTW, HISTW)])


def _tc_finish(h_ref, o_ref):
    h = h_ref[:]
    s = jnp.sum(h, axis=0)
    loss = 0.0
    for b in range(NBINS):
        loss = loss + jnp.abs(jnp.sum(s[b * 16:(b + 1) * 16]))
    o_ref[:, :] = jnp.full((1, 128), loss, jnp.float32)


def kernel(inputs, targets):
    N, C = inputs.shape
    R = min(32768, N)
    NB = N // R
    T = R // 128
    tgt3 = targets.astype(jnp.int32).reshape(NB, 1, R)
    d_arr, b_arr = pl.pallas_call(
        _tc_rows_block,
        grid=(NB,),
        in_specs=[
            pl.BlockSpec((R, C), lambda i: (i, 0)),
            pl.BlockSpec((1, 1, R), lambda i: (i, 0, 0)),
        ],
        out_specs=[
            pl.BlockSpec((1, T, 128), lambda i: (i, 0, 0)),
            pl.BlockSpec((1, T, 128), lambda i: (i, 0, 0)),
        ],
        out_shape=[
            jax.ShapeDtypeStruct((NB, T, 128), jnp.float32),
            jax.ShapeDtypeStruct((NB, T, 128), jnp.int32),
        ],
    )(inputs, tgt3)
    d_flat = d_arr.reshape(N)
    b_flat = b_arr.reshape(N)

    chunk = N // NWORKERS
    sc_fn = functools.partial(
        pl.kernel,
        out_type=jax.ShapeDtypeStruct((NWORKERS * HISTW,), jnp.float32),
        mesh=plsc.VectorSubcoreMesh(core_axis_name="c", subcore_axis_name="s"),
        scratch_types=[
            pltpu.VMEM((chunk,), jnp.float32),
            pltpu.VMEM((chunk,), jnp.int32),
            pltpu.VMEM((HISTW,), jnp.float32),
        ],
    )(_sc_hist)
    hist = sc_fn(d_flat, b_flat).reshape(NWORKERS, HISTW)

    out = pl.pallas_call(
        _tc_finish,
        out_specs=pl.BlockSpec((1, 128), lambda: (0, 0)),
        out_shape=jax.ShapeDtypeStruct((1, 128), jnp.float32),
    )(hist)
    return (out[0, 0] / N).reshape(())
